# Initial kernel scaffold; baseline (speedup 1.0000x reference)
import jax
import jax.numpy as jnp
from jax.experimental import pallas as pl


def kernel(visit_order, visit_rel_times, pos_table, time_table):
    raise NotImplementedError("write your pallas kernel here")


# speedup vs baseline: 4.2546x; 4.2546x over previous
import functools

import jax
import jax.numpy as jnp
from jax import lax
from jax.experimental import pallas as pl
from jax.experimental.pallas import tpu as pltpu
from jax.experimental.pallas import tpu_sc as plsc

_BATCH = 4096
_HIST = 200
_EMBED = 64
_B = _BATCH * _HIST
_NW = 32
_B_PER_W = _B // _NW
_C = 800
_NPAIR = _B_PER_W // (2 * _C)


def _make_gather():
    mesh = plsc.VectorSubcoreMesh(core_axis_name="c", subcore_axis_name="s")

    @functools.partial(
        pl.kernel,
        mesh=mesh,
        out_type=jax.ShapeDtypeStruct((_B, _EMBED), jnp.float32),
        scratch_types=[
            pltpu.VMEM((_B_PER_W,), jnp.int32),
            pltpu.VMEM((_C, _EMBED), jnp.float32),
            pltpu.VMEM((_C, _EMBED), jnp.float32),
            pltpu.SemaphoreType.DMA,
            pltpu.SemaphoreType.DMA,
        ],
        compiler_params=pltpu.CompilerParams(use_tc_tiling_on_sc=False),
    )
    def gather_kernel(idx_hbm, table_hbm, out_hbm, idx_v, rows0, rows1,
                      gsem0, gsem1):
        wid = lax.axis_index("s") * 2 + lax.axis_index("c")
        base = wid * _B_PER_W
        pltpu.sync_copy(idx_hbm.at[pl.ds(base, _B_PER_W)], idx_v)

        def gather_start(g, rows, sem):
            pltpu.async_copy(
                table_hbm.at[idx_v.at[pl.ds(g * _C, _C)]], rows, sem)

        def gather_wait(g, rows, sem):
            pltpu.make_async_copy(
                table_hbm.at[idx_v.at[pl.ds(g * _C, _C)]], rows, sem).wait()

        def out_sync(g, rows):
            pltpu.sync_copy(rows, out_hbm.at[pl.ds(base + g * _C, _C)])

        gather_start(0, rows0, gsem0)

        def body(i, carry):
            g = 2 * i
            gather_start(g + 1, rows1, gsem1)
            gather_wait(g, rows0, gsem0)
            out_sync(g, rows0)

            @pl.when(i + 1 < _NPAIR)
            def _():
                gather_start(g + 2, rows0, gsem0)

            gather_wait(g + 1, rows1, gsem1)
            out_sync(g + 1, rows1)
            return carry

        lax.fori_loop(0, _NPAIR, body, 0)

    return gather_kernel


_gather = _make_gather()


def kernel(visit_order, visit_rel_times, pos_table, time_table):
    idx = visit_rel_times.reshape(_B).astype(jnp.int32)
    out = _gather(idx, time_table)
    return out.reshape(_BATCH, _HIST, _EMBED)


# speedup vs baseline: 4.2628x; 1.0019x over previous
import functools

import jax
import jax.numpy as jnp
from jax import lax
from jax.experimental import pallas as pl
from jax.experimental.pallas import tpu as pltpu
from jax.experimental.pallas import tpu_sc as plsc

_BATCH = 4096
_HIST = 200
_EMBED = 64
_B = _BATCH * _HIST
_NW = 32
_B_PER_W = _B // _NW
_BAT_PER_W = _BATCH // _NW
_CB = 4
_C = _CB * _HIST
_NPAIR = _B_PER_W // (2 * _C)


def _make_gather():
    mesh = plsc.VectorSubcoreMesh(core_axis_name="c", subcore_axis_name="s")

    @functools.partial(
        pl.kernel,
        mesh=mesh,
        out_type=jax.ShapeDtypeStruct((_BATCH, _HIST, _EMBED), jnp.float32),
        scratch_types=[
            pltpu.VMEM((_B_PER_W,), jnp.int32),
            pltpu.VMEM((_C, _EMBED), jnp.float32),
            pltpu.VMEM((_C, _EMBED), jnp.float32),
            pltpu.SemaphoreType.DMA,
            pltpu.SemaphoreType.DMA,
            pltpu.SemaphoreType.DMA,
        ],
        compiler_params=pltpu.CompilerParams(use_tc_tiling_on_sc=False),
    )
    def gather_kernel(idx_hbm, table_hbm, out_hbm, idx_v, rows0, rows1,
                      gsem0, gsem1, osem):
        wid = lax.axis_index("s") * 2 + lax.axis_index("c")
        base = wid * _B_PER_W
        bat_base = wid * _BAT_PER_W
        pltpu.sync_copy(idx_hbm.at[pl.ds(base, _B_PER_W)], idx_v)

        def gather_start(g, rows, sem):
            pltpu.async_copy(
                table_hbm.at[idx_v.at[pl.ds(g * _C, _C)]], rows, sem)

        def gather_wait(g, rows, sem):
            pltpu.make_async_copy(
                table_hbm.at[idx_v.at[pl.ds(g * _C, _C)]], rows, sem).wait()

        def out_start(g, rows):
            for j in range(_CB):
                pltpu.async_copy(
                    rows.at[pl.ds(j * _HIST, _HIST)],
                    out_hbm.at[bat_base + g * _CB + j], osem)

        def out_wait(g, rows):
            for j in range(_CB):
                pltpu.make_async_copy(
                    rows.at[pl.ds(j * _HIST, _HIST)],
                    out_hbm.at[bat_base + g * _CB + j], osem).wait()

        gather_start(0, rows0, gsem0)

        def body(i, carry):
            g = 2 * i
            gather_start(g + 1, rows1, gsem1)
            gather_wait(g, rows0, gsem0)
            out_start(g, rows0)
            out_wait(g, rows0)

            @pl.when(i + 1 < _NPAIR)
            def _():
                gather_start(g + 2, rows0, gsem0)

            gather_wait(g + 1, rows1, gsem1)
            out_start(g + 1, rows1)
            out_wait(g + 1, rows1)
            return carry

        lax.fori_loop(0, _NPAIR, body, 0)

    return gather_kernel


_gather = _make_gather()


def kernel(visit_order, visit_rel_times, pos_table, time_table):
    idx = visit_rel_times.reshape(_B).astype(jnp.int32)
    return _gather(idx, time_table)
